# Initial kernel scaffold; baseline (speedup 1.0000x reference)
import functools

import jax
import jax.numpy as jnp
from jax import lax
from jax.experimental import pallas as pl
from jax.experimental.pallas import tpu as pltpu
from jax.experimental.pallas import tpu_sc as plsc

_B, _H, _N, _DH = 8, 12, 577, 64
_K = 256
_KO = _K + 1
_NM = _N - 1
_KP = 272
_EPS = 1e-06
_MASK_VAL = -jnp.finfo(jnp.float32).max / 2


def _sample_body(cls_ref, val_ref, gum_ref, msk_ref, uid_ref, nm_ref):
    v = val_ref[0]
    vn = jnp.sqrt(jnp.sum(v * v, axis=-1))
    ca = jnp.sum(cls_ref[0] * vn, axis=0, keepdims=True)
    normed = ca / (jnp.sum(ca) + _EPS)
    logits = jnp.log(normed + _EPS)
    logits = jnp.where(msk_ref[0] > 0, logits, _MASK_VAL)
    scores = logits + gum_ref[0]
    am = jnp.argmax(scores, axis=1, keepdims=True)
    n_iota = lax.broadcasted_iota(jnp.int32, (_K, _NM), 1)
    member = jnp.any(am == n_iota, axis=0, keepdims=True)
    memf = member.astype(jnp.float32)
    m_i = lax.broadcasted_iota(jnp.int32, (_NM, _NM), 0)
    n_i = lax.broadcasted_iota(jnp.int32, (_NM, _NM), 1)
    tril = (m_i <= n_i).astype(jnp.float32)
    rank = jnp.dot(memf, tril, preferred_element_type=jnp.float32)
    ranki = rank.astype(jnp.int32)
    count = jnp.sum(member.astype(jnp.int32))
    i_iota = lax.broadcasted_iota(jnp.int32, (_KP, _NM), 0)
    n_iota2 = lax.broadcasted_iota(jnp.int32, (_KP, _NM), 1)
    sel = (ranki == i_iota) & member
    uid = jnp.sum(jnp.where(sel, n_iota2 + 1, 0), axis=1, keepdims=True)
    uid_ref[0] = uid
    io = lax.broadcasted_iota(jnp.int32, (_KO, 1), 0)
    nm_ref[0] = (io <= count).astype(jnp.int32)


def _sample_ids(cls_attn, value_t, gumbel, maskf):
    return pl.pallas_call(
        _sample_body,
        grid=(_B,),
        in_specs=[
            pl.BlockSpec((1, _H, _NM), lambda b: (b, 0, 0)),
            pl.BlockSpec((1, _H, _NM, _DH), lambda b: (b, 0, 0, 0)),
            pl.BlockSpec((1, _K, _NM), lambda b: (b, 0, 0)),
            pl.BlockSpec((1, 1, _NM), lambda b: (b, 0, 0)),
        ],
        out_specs=[
            pl.BlockSpec((1, _KP, 1), lambda b: (b, 0, 0)),
            pl.BlockSpec((1, _KO, 1), lambda b: (b, 0, 0)),
        ],
        out_shape=[
            jax.ShapeDtypeStruct((_B, _KP, 1), jnp.int32),
            jax.ShapeDtypeStruct((_B, _KO, 1), jnp.int32),
        ],
    )(cls_attn, value_t, gumbel, maskf)


_INFO = plsc.get_sparse_core_info()
_NC, _NS = _INFO.num_cores, _INFO.num_subcores
_NW = _NC * _NS
_PAIRS = _B * _H
_PPW = _PAIRS // _NW
_CHUNKS = ((0, 88), (88, 88), (176, 81))


@functools.partial(
    pl.kernel,
    mesh=plsc.VectorSubcoreMesh(core_axis_name="c", subcore_axis_name="s"),
    out_type=jax.ShapeDtypeStruct((_PAIRS, _KO, _N), jnp.float32),
    scratch_types=[
        pltpu.VMEM((_KP,), jnp.int32),
        pltpu.VMEM((88, _N), jnp.float32),
        pltpu.SemaphoreType.DMA,
    ],
)
def _sc_gather(table_hbm, ids_hbm, out_hbm, idx_v, buf, sem):
    wid = lax.axis_index("s") * _NC + lax.axis_index("c")
    for p in range(_PPW):
        pair = wid * _PPW + p
        b = pair // _H
        pltpu.sync_copy(ids_hbm.at[b], idx_v)
        base = pair * _N
        for i in range(_KP // 16):
            sl = pl.ds(i * 16, 16)
            idx_v[sl] = idx_v[sl] + base
        for c0, cn in _CHUNKS:
            cp = pltpu.async_copy(
                table_hbm.at[idx_v.at[pl.ds(c0, cn)]], buf.at[pl.ds(0, cn)], sem)
            cp.wait()
            pltpu.sync_copy(buf.at[pl.ds(0, cn)], out_hbm.at[pair, pl.ds(c0, cn)])


def kernel(attn, value, mask):
    u = jax.random.uniform(jax.random.key(42), (_B, _K, _NM),
                           dtype=attn.dtype, minval=0.0, maxval=1.0)
    gumbel = -jnp.log(-jnp.log(u + _EPS) + _EPS)
    cls_attn = attn[:, :, 0, 1:]
    value_t = value[:, :, 1:, :]
    maskf = mask[:, 1:].astype(jnp.float32).reshape(_B, 1, _NM)

    uid_out, nm_out = _sample_ids(cls_attn, value_t, gumbel, maskf)
    uidc = uid_out[:, :, 0]
    unique_ids = uidc[:, :_KO]
    new_mask = nm_out[:, :, 0] != 0

    table = attn.reshape(_B * _H * _N, _N)
    new_attn = _sc_gather(table, uidc).reshape(_B, _H, _KO, _N)
    return new_attn, new_mask, unique_ids


# speedup vs baseline: 1.6333x; 1.6333x over previous
import functools

import jax
import jax.numpy as jnp
from jax import lax
from jax.experimental import pallas as pl
from jax.experimental.pallas import tpu as pltpu
from jax.experimental.pallas import tpu_sc as plsc

_B, _H, _N, _DH = 8, 12, 577, 64
_K = 256
_KO = _K + 1
_NM = _N - 1
_KP = 272
_EPS = 1e-06
_MASK_VAL = -jnp.finfo(jnp.float32).max / 2


def _sample_body(cls_ref, val_ref, gum_ref, msk_ref, uid_ref, nm_ref):
    v = val_ref[0]
    vn = jnp.sqrt(jnp.sum(v * v, axis=-1))
    ca = jnp.sum(cls_ref[0] * vn, axis=0, keepdims=True)
    normed = ca / (jnp.sum(ca) + _EPS)
    logits = jnp.log(normed + _EPS)
    logits = jnp.where(msk_ref[0] > 0, logits, _MASK_VAL)
    scores = logits + gum_ref[0]
    am = jnp.argmax(scores, axis=1, keepdims=True)
    n_iota = lax.broadcasted_iota(jnp.int32, (_K, _NM), 1)
    member = jnp.any(am == n_iota, axis=0, keepdims=True)
    memf = member.astype(jnp.float32)
    m_i = lax.broadcasted_iota(jnp.int32, (_NM, _NM), 0)
    n_i = lax.broadcasted_iota(jnp.int32, (_NM, _NM), 1)
    tril = (m_i <= n_i).astype(jnp.float32)
    rank = jnp.dot(memf, tril, preferred_element_type=jnp.float32)
    ranki = rank.astype(jnp.int32)
    count = jnp.sum(member.astype(jnp.int32))
    i_iota = lax.broadcasted_iota(jnp.int32, (_KP, _NM), 0)
    n_iota2 = lax.broadcasted_iota(jnp.int32, (_KP, _NM), 1)
    sel = (ranki == i_iota) & member
    uid = jnp.sum(jnp.where(sel, n_iota2 + 1, 0), axis=1, keepdims=True)
    uid_ref[0] = uid
    io = lax.broadcasted_iota(jnp.int32, (_KO, 1), 0)
    nm_ref[0] = (io <= count).astype(jnp.int32)


def _sample_ids(cls_attn, value_t, gumbel, maskf):
    return pl.pallas_call(
        _sample_body,
        grid=(_B,),
        in_specs=[
            pl.BlockSpec((1, _H, _NM), lambda b: (b, 0, 0)),
            pl.BlockSpec((1, _H, _NM, _DH), lambda b: (b, 0, 0, 0)),
            pl.BlockSpec((1, _K, _NM), lambda b: (b, 0, 0)),
            pl.BlockSpec((1, 1, _NM), lambda b: (b, 0, 0)),
        ],
        out_specs=[
            pl.BlockSpec((1, _KP, 1), lambda b: (b, 0, 0)),
            pl.BlockSpec((1, _KO, 1), lambda b: (b, 0, 0)),
        ],
        out_shape=[
            jax.ShapeDtypeStruct((_B, _KP, 1), jnp.int32),
            jax.ShapeDtypeStruct((_B, _KO, 1), jnp.int32),
        ],
    )(cls_attn, value_t, gumbel, maskf)


_NC, _NS = 2, 16
_NW = _NC * _NS
_PAIRS = _B * _H
_PPW = _PAIRS // _NW
_CHUNKS = ((0, 88), (88, 88), (176, 81))


@functools.cache
def _make_sc_gather():
    @functools.partial(
        pl.kernel,
        mesh=plsc.VectorSubcoreMesh(core_axis_name="c", subcore_axis_name="s",
                                    num_cores=_NC, num_subcores=_NS),
        out_type=jax.ShapeDtypeStruct((_PAIRS, _KO, _N), jnp.float32),
        scratch_types=[
            pltpu.VMEM((_KP,), jnp.int32),
            pltpu.VMEM((88, _N), jnp.float32),
            pltpu.VMEM((81, _N), jnp.float32),
            pltpu.SemaphoreType.DMA,
        ],
        compiler_params=pltpu.CompilerParams(use_tc_tiling_on_sc=False),
    )
    def _sc_gather(table_hbm, ids_hbm, out_hbm, idx_v, buf_a, buf_c, sem):
        wid = lax.axis_index("s") * _NC + lax.axis_index("c")
        bufs = (buf_a, buf_a, buf_c)
        for p in range(_PPW):
            pair = wid * _PPW + p
            b = pair // _H
            pltpu.sync_copy(ids_hbm.at[b], idx_v)
            base = pair * _N
            for i in range(_KP // 16):
                sl = pl.ds(i * 16, 16)
                idx_v[sl] = idx_v[sl] + base
            for (c0, cn), buf in zip(_CHUNKS, bufs):
                cp = pltpu.async_copy(
                    table_hbm.at[idx_v.at[pl.ds(c0, cn)]], buf, sem)
                cp.wait()
                pltpu.sync_copy(buf, out_hbm.at[pair, pl.ds(c0, cn)])

    return _sc_gather


def kernel(attn, value, mask):
    u = jax.random.uniform(jax.random.key(42), (_B, _K, _NM),
                           dtype=attn.dtype, minval=0.0, maxval=1.0)
    gumbel = -jnp.log(-jnp.log(u + _EPS) + _EPS)
    cls_attn = attn[:, :, 0, 1:]
    value_t = value[:, :, 1:, :]
    maskf = mask[:, 1:].astype(jnp.float32).reshape(_B, 1, _NM)

    uid_out, nm_out = _sample_ids(cls_attn, value_t, gumbel, maskf)
    uidc = uid_out[:, :, 0]
    unique_ids = uidc[:, :_KO]
    new_mask = nm_out[:, :, 0] != 0

    table = attn.reshape(_B * _H * _N, _N)
    new_attn = _make_sc_gather()(table, uidc).reshape(_B, _H, _KO, _N)
    return new_attn, new_mask, unique_ids


# speedup vs baseline: 5.6584x; 3.4643x over previous
import functools

import jax
import jax.numpy as jnp
from jax import lax
from jax.experimental import pallas as pl
from jax.experimental.pallas import tpu as pltpu
from jax.experimental.pallas import tpu_sc as plsc

_B, _H, _N, _DH = 8, 12, 577, 64
_K = 256
_KO = _K + 1
_NM = _N - 1
_KP = 272
_EPS = 1e-06
_MASK_VAL = -jnp.finfo(jnp.float32).max / 2


def _sample_body(cls_ref, val_ref, gum_ref, msk_ref, uid_ref, nm_ref):
    v = val_ref[0]
    vn = jnp.sqrt(jnp.sum(v * v, axis=-1))
    ca = jnp.sum(cls_ref[0] * vn, axis=0, keepdims=True)
    normed = ca / (jnp.sum(ca) + _EPS)
    logits = jnp.log(normed + _EPS)
    logits = jnp.where(msk_ref[0] > 0, logits, _MASK_VAL)
    scores = logits + gum_ref[0]
    am = jnp.argmax(scores, axis=1, keepdims=True)
    n_iota = lax.broadcasted_iota(jnp.int32, (_K, _NM), 1)
    member = jnp.any(am == n_iota, axis=0, keepdims=True)
    memf = member.astype(jnp.float32)
    m_i = lax.broadcasted_iota(jnp.int32, (_NM, _NM), 0)
    n_i = lax.broadcasted_iota(jnp.int32, (_NM, _NM), 1)
    tril = (m_i <= n_i).astype(jnp.float32)
    rank = jnp.dot(memf, tril, preferred_element_type=jnp.float32)
    ranki = rank.astype(jnp.int32)
    count = jnp.sum(member.astype(jnp.int32))
    i_iota = lax.broadcasted_iota(jnp.int32, (_KP, _NM), 0)
    n_iota2 = lax.broadcasted_iota(jnp.int32, (_KP, _NM), 1)
    sel = (ranki == i_iota) & member
    uid = jnp.sum(jnp.where(sel, n_iota2 + 1, 0), axis=1, keepdims=True)
    uid_ref[0] = uid
    io = lax.broadcasted_iota(jnp.int32, (_KO, 1), 0)
    nm_ref[0] = (io <= count).astype(jnp.int32)


def _sample_ids(cls_attn, value_t, gumbel, maskf):
    return pl.pallas_call(
        _sample_body,
        grid=(_B,),
        in_specs=[
            pl.BlockSpec((1, _H, _NM), lambda b: (b, 0, 0)),
            pl.BlockSpec((1, _H, _NM, _DH), lambda b: (b, 0, 0, 0)),
            pl.BlockSpec((1, _K, _NM), lambda b: (b, 0, 0)),
            pl.BlockSpec((1, 1, _NM), lambda b: (b, 0, 0)),
        ],
        out_specs=[
            pl.BlockSpec((1, _KP, 1), lambda b: (b, 0, 0)),
            pl.BlockSpec((1, _KO, 1), lambda b: (b, 0, 0)),
        ],
        out_shape=[
            jax.ShapeDtypeStruct((_B, _KP, 1), jnp.int32),
            jax.ShapeDtypeStruct((_B, _KO, 1), jnp.int32),
        ],
    )(cls_attn, value_t, gumbel, maskf)


_NC, _NS = 2, 16
_NW = _NC * _NS
_PAIRS = _B * _H
_PPW = _PAIRS // _NW
_CHUNKS = ((0, 88), (88, 88), (176, 81))


def _gather_tc_body(uid_ref, attn_ref, out_ref):
    ids = uid_ref[0]
    n_iota = lax.broadcasted_iota(jnp.int32, (_KP, _N), 1)
    sel = (ids == n_iota).astype(jnp.float32)
    slab = attn_ref[0, 0]
    rows = jnp.dot(sel, slab, preferred_element_type=jnp.float32)
    out_ref[0, 0] = rows[:_KO, :]


def _tc_gather(uid3, attn):
    return pl.pallas_call(
        _gather_tc_body,
        grid=(_B, _H),
        in_specs=[
            pl.BlockSpec((1, _KP, 1), lambda b, h: (b, 0, 0)),
            pl.BlockSpec((1, 1, _N, _N), lambda b, h: (b, h, 0, 0)),
        ],
        out_specs=pl.BlockSpec((1, 1, _KO, _N), lambda b, h: (b, h, 0, 0)),
        out_shape=jax.ShapeDtypeStruct((_B, _H, _KO, _N), jnp.float32),
    )(uid3, attn)


@functools.cache
def _make_sc_gather():
    @functools.partial(
        pl.kernel,
        mesh=plsc.VectorSubcoreMesh(core_axis_name="c", subcore_axis_name="s",
                                    num_cores=_NC, num_subcores=_NS),
        out_type=jax.ShapeDtypeStruct((_PAIRS, _KO, _N), jnp.float32),
        scratch_types=[
            pltpu.VMEM((_KP,), jnp.int32),
            pltpu.VMEM((88, _N), jnp.float32),
            pltpu.VMEM((81, _N), jnp.float32),
            pltpu.SemaphoreType.DMA,
        ],
        compiler_params=pltpu.CompilerParams(use_tc_tiling_on_sc=False),
    )
    def _sc_gather(table_hbm, ids_hbm, out_hbm, idx_v, buf_a, buf_c, sem):
        wid = lax.axis_index("s") * _NC + lax.axis_index("c")
        bufs = (buf_a, buf_a, buf_c)
        for p in range(_PPW):
            pair = wid * _PPW + p
            b = pair // _H
            pltpu.sync_copy(ids_hbm.at[b], idx_v)
            base = pair * _N
            for i in range(_KP // 16):
                sl = pl.ds(i * 16, 16)
                idx_v[sl] = idx_v[sl] + base
            for (c0, cn), buf in zip(_CHUNKS, bufs):
                cp = pltpu.async_copy(
                    table_hbm.at[idx_v.at[pl.ds(c0, cn)]], buf, sem)
                cp.wait()
                pltpu.sync_copy(buf, out_hbm.at[pair, pl.ds(c0, cn)])

    return _sc_gather


def kernel(attn, value, mask):
    u = jax.random.uniform(jax.random.key(42), (_B, _K, _NM),
                           dtype=attn.dtype, minval=0.0, maxval=1.0)
    gumbel = -jnp.log(-jnp.log(u + _EPS) + _EPS)
    cls_attn = attn[:, :, 0, 1:]
    value_t = value[:, :, 1:, :]
    maskf = mask[:, 1:].astype(jnp.float32).reshape(_B, 1, _NM)

    uid_out, nm_out = _sample_ids(cls_attn, value_t, gumbel, maskf)
    uidc = uid_out[:, :, 0]
    unique_ids = uidc[:, :_KO]
    new_mask = nm_out[:, :, 0] != 0

    new_attn = _tc_gather(uid_out, attn)
    return new_attn, new_mask, unique_ids
